# Initial kernel scaffold; baseline (speedup 1.0000x reference)
import jax
import jax.numpy as jnp
from jax.experimental import pallas as pl


def kernel(user_id, pos_id, neg_id, user_table, item_table, train_label):
    raise NotImplementedError("write your pallas kernel here")


# speedup vs baseline: 1.7509x; 1.7509x over previous
import jax
import jax.numpy as jnp
from jax.experimental import pallas as pl
from jax.experimental.pallas import tpu as pltpu


def _mf_kernel(uid_ref,
               user_hbm, label_hbm,
               item_ref,
               pid_ref, nid_ref,
               user_out, pos_out, neg_out, com_out,
               bl_buf, sems):
    b = pl.program_id(0)
    bt, num_items = bl_buf.shape
    b0 = b * bt

    for j in range(bt):
        u = uid_ref[b0 + j]
        pltpu.make_async_copy(
            label_hbm.at[pl.ds(u, 1), :], bl_buf.at[pl.ds(j, 1), :],
            sems.at[0]).start()
        pltpu.make_async_copy(
            user_hbm.at[pl.ds(u, 1), :], user_out.at[pl.ds(j, 1), :],
            sems.at[1]).start()

    item = item_ref[...]

    lane = jax.lax.broadcasted_iota(jnp.int32, (bt, num_items), 1)
    oh_pos = (pid_ref[...] == lane).astype(jnp.float32)
    oh_neg = (nid_ref[...] == lane).astype(jnp.float32)
    pos_out[...] = jnp.dot(oh_pos, item, preferred_element_type=jnp.float32)
    neg_out[...] = jnp.dot(oh_neg, item, preferred_element_type=jnp.float32)

    pltpu.make_async_copy(
        label_hbm.at[pl.ds(0, bt), :], bl_buf.at[pl.ds(0, bt), :],
        sems.at[0]).wait()

    bl = bl_buf[...]
    acc = jnp.dot(bl, item, preferred_element_type=jnp.float32)
    num = jnp.sum(bl, axis=1, keepdims=True)
    com_out[...] = acc / jnp.where(num > 0.0, num, 1.0)

    pltpu.make_async_copy(
        user_hbm.at[pl.ds(0, bt), :], user_out.at[pl.ds(0, bt), :],
        sems.at[1]).wait()


def kernel(user_id, pos_id, neg_id, user_table, item_table, train_label):
    bt = 256
    B = user_id.shape[0]
    num_users, dim = user_table.shape
    num_items = item_table.shape[0]

    nb = pl.cdiv(B, bt)
    Bp = nb * bt
    pad = Bp - B

    uid = jnp.clip(jnp.pad(user_id.astype(jnp.int32), (0, pad)), 0, num_users - 1)
    pid = jnp.clip(jnp.pad(pos_id.astype(jnp.int32), (0, pad)), 0, num_items - 1)
    nid = jnp.clip(jnp.pad(neg_id.astype(jnp.int32), (0, pad)), 0, num_items - 1)
    pid2 = pid.reshape(Bp, 1)
    nid2 = nid.reshape(Bp, 1)

    grid_spec = pltpu.PrefetchScalarGridSpec(
        num_scalar_prefetch=1,
        grid=(nb,),
        in_specs=[
            pl.BlockSpec(memory_space=pl.ANY),
            pl.BlockSpec(memory_space=pl.ANY),
            pl.BlockSpec((num_items, dim), lambda b, uid: (0, 0)),
            pl.BlockSpec((bt, 1), lambda b, uid: (b, 0)),
            pl.BlockSpec((bt, 1), lambda b, uid: (b, 0)),
        ],
        out_specs=[pl.BlockSpec((bt, dim), lambda b, uid: (b, 0))] * 4,
        scratch_shapes=[
            pltpu.VMEM((bt, num_items), jnp.float32),
            pltpu.SemaphoreType.DMA((2,)),
        ],
    )

    outs = pl.pallas_call(
        _mf_kernel,
        out_shape=[jax.ShapeDtypeStruct((Bp, dim), jnp.float32)] * 4,
        grid_spec=grid_spec,
        compiler_params=pltpu.CompilerParams(
            dimension_semantics=("parallel",),
            vmem_limit_bytes=64 * 1024 * 1024),
    )(uid,
      user_table.astype(jnp.float32),
      train_label.astype(jnp.float32),
      item_table.astype(jnp.float32),
      pid2, nid2)

    if pad:
        outs = [o[:B] for o in outs]
    return tuple(outs)
